# baseline (speedup 1.0000x reference)
import functools

import jax
import jax.numpy as jnp
from jax import lax
from jax.experimental import pallas as pl
from jax.experimental.pallas import tpu as pltpu
from jax.experimental.pallas import tpu_sc as plsc

_L = 16
_B = 128
_K = 2048


def _floor_i32(x):
    t = x.astype(jnp.int32)
    return jnp.where(x < t.astype(jnp.float32), t - 1, t)


def _feature_level_sc(ux, uy, g0f, t1, n, res0, res1, c0, c1):
    nworkers = 32
    per_w = n // nworkers
    steps = per_w // _B
    cout = 4 * c0 + c1
    npos = res0 * res0
    plane = npos
    pos_per_tile = npos // 16
    blocks = pos_per_tile // _K

    mesh = plsc.VectorSubcoreMesh(core_axis_name="c", subcore_axis_name="s")

    @functools.partial(
        pl.kernel,
        mesh=mesh,
        compiler_params=pltpu.CompilerParams(use_tc_tiling_on_sc=False,
                                             needs_layout_passes=False),
        out_type=jax.ShapeDtypeStruct((n, cout), jnp.float32),
        scratch_types=[
            pltpu.VMEM((_B,), jnp.float32),
            pltpu.VMEM((_B,), jnp.float32),
            pltpu.VMEM((8, _B), jnp.int32),
            pltpu.VMEM((4, _B), jnp.float32),
            pltpu.VMEM((_B, 8), jnp.float32),
            pltpu.VMEM((_B, 8), jnp.float32),
            pltpu.VMEM((_B, 8), jnp.float32),
            pltpu.VMEM((_B, 8), jnp.float32),
            pltpu.VMEM((_B, 16), jnp.float32),
            pltpu.VMEM((_B, 16), jnp.float32),
            pltpu.VMEM((_B, 16), jnp.float32),
            pltpu.VMEM((_B, 16), jnp.float32),
            pltpu.VMEM((_B, 48), jnp.float32),
            pltpu.VMEM((8, _K), jnp.float32),
            pltpu.VMEM((8, _K), jnp.float32),
            pltpu.VMEM((_K, 8), jnp.float32),
            pltpu.VMEM((_K, 8), jnp.float32),
            pltpu.HBM((2, npos, 8), jnp.float32),
            pltpu.SemaphoreType.DMA,
            pltpu.SemaphoreType.DMA,
            pltpu.SemaphoreType.DMA,
            pltpu.SemaphoreType.DMA,
            pltpu.SemaphoreType.DMA,
        ],
    )
    def k(ux_hbm, uy_hbm, g0_hbm, t1_hbm, out_hbm,
          uxv, uyv, idx, wts, c00, c01, c10, c11, s00, s01, s10, s11,
          outv, tp0, tp1, to0, to1, tbl, sem, isem0, isem1, osem0, osem1):
        sid = lax.axis_index("s")
        cc = lax.axis_index("c")
        wid = sid * 2 + cc

        def in_src(blk):
            return g0_hbm.at[:, pl.ds(sid * pos_per_tile + blk * _K, _K)]

        def out_dst(blk):
            return tbl.at[cc, pl.ds(sid * pos_per_tile + blk * _K, _K)]

        pltpu.async_copy(in_src(0), tp0, isem0)
        pltpu.async_copy(in_src(1), tp1, isem1)

        def tpair(b2, _):
            for par, (tpb, tob, isem, osem) in enumerate(
                    ((tp0, to0, isem0, osem0), (tp1, to1, isem1, osem1))):
                blk = b2 * 2 + par
                pltpu.make_async_copy(in_src(blk), tpb, isem).wait()

                @pl.when(blk >= 2)
                def _():
                    pltpu.make_async_copy(tob, out_dst(blk - 2), osem).wait()

                def shuffle(q, _):
                    lane = lax.iota(jnp.int32, _L)
                    drows = q * _L + lane
                    for c in range(8):
                        v = tpb[c, pl.ds(q * _L, _L)]
                        cv = jnp.full((_L,), c, jnp.int32)
                        plsc.store_scatter(tob, [drows, cv], v)
                    return 0

                lax.fori_loop(0, _K // _L, shuffle, 0)
                pltpu.async_copy(tob, out_dst(blk), osem)

                @pl.when(blk + 2 < blocks)
                def _():
                    pltpu.async_copy(in_src(blk + 2), tpb, isem)
            return 0

        lax.fori_loop(0, blocks // 2, tpair, 0)
        pltpu.make_async_copy(to0, out_dst(blocks - 2), osem0).wait()
        pltpu.make_async_copy(to1, out_dst(blocks - 1), osem1).wait()
        plsc.subcore_barrier()

        def step(st, _):
            base = wid * per_w + st * _B
            pltpu.sync_copy(ux_hbm.at[pl.ds(base, _B)], uxv)
            pltpu.sync_copy(uy_hbm.at[pl.ds(base, _B)], uyv)

            for g in range(_B // _L):
                sl = pl.ds(g * _L, _L)
                x = uxv[sl]
                y = uyv[sl]
                fx = _floor_i32(x * res0 - 0.5)
                fy = _floor_i32(y * res0 - 0.5)
                x0 = jnp.clip(fx, 0, res0 - 2)
                y0 = jnp.clip(fy, 0, res0 - 2)
                b00 = y0 * res0 + x0
                lane = lax.iota(jnp.int32, _L)
                r_a, r_b = (0, 2) if g < 4 else (1, 3)
                posv = 2 * ((g % 4) * _L) + 2 * lane
                plsc.store_scatter(idx.at[r_a], [posv], b00)
                plsc.store_scatter(idx.at[r_a], [posv + 1], b00 + 1)
                plsc.store_scatter(idx.at[r_b], [posv], b00 + res0)
                plsc.store_scatter(idx.at[r_b], [posv + 1], b00 + res0 + 1)
                qx = x * res1 - 0.5
                qy = y * res1 - 0.5
                ix0 = _floor_i32(qx)
                iy0 = _floor_i32(qy)
                wx1 = qx - ix0.astype(jnp.float32)
                wy1 = qy - iy0.astype(jnp.float32)
                wx0 = 1.0 - wx1
                wy0 = 1.0 - wy1
                wx0 = jnp.where(ix0 >= 0, wx0, 0.0)
                wy0 = jnp.where(iy0 >= 0, wy0, 0.0)
                wx1 = jnp.where(ix0 + 1 <= res1 - 1, wx1, 0.0)
                wy1 = jnp.where(iy0 + 1 <= res1 - 1, wy1, 0.0)
                jx0 = jnp.maximum(ix0, 0)
                jy0 = jnp.maximum(iy0, 0)
                jx1 = jnp.minimum(ix0 + 1, res1 - 1)
                jy1 = jnp.minimum(iy0 + 1, res1 - 1)
                idx[4, sl] = jy0 * res1 + jx0
                idx[5, sl] = jy0 * res1 + jx1
                idx[6, sl] = jy1 * res1 + jx0
                idx[7, sl] = jy1 * res1 + jx1
                wts[0, sl] = wy0 * wx0
                wts[1, sl] = wy0 * wx1
                wts[2, sl] = wy1 * wx0
                wts[3, sl] = wy1 * wx1

            cps = [
                pltpu.async_copy(tbl.at[cc].at[idx.at[0]], c00, sem),
                pltpu.async_copy(tbl.at[cc].at[idx.at[1]], c01, sem),
                pltpu.async_copy(tbl.at[cc].at[idx.at[2]], c10, sem),
                pltpu.async_copy(tbl.at[cc].at[idx.at[3]], c11, sem),
                pltpu.async_copy(t1_hbm.at[idx.at[4]], s00, sem),
                pltpu.async_copy(t1_hbm.at[idx.at[5]], s01, sem),
                pltpu.async_copy(t1_hbm.at[idx.at[6]], s10, sem),
                pltpu.async_copy(t1_hbm.at[idx.at[7]], s11, sem),
            ]
            for cp in cps:
                cp.wait()

            for g in range(_B // _L):
                g16 = g * _L
                lane = lax.iota(jnp.int32, _L)
                colsrc = lane & 7
                ca, cb = (c00, c10) if g < 4 else (c01, c11)
                for t in range(_L):
                    il = (g % 4) * _L + t
                    prow = 2 * il + (lane >> 3)
                    outv[g16 + t, 0:16] = plsc.load_gather(ca, [prow, colsrc])
                    outv[g16 + t, 16:32] = plsc.load_gather(cb, [prow, colsrc])
                rows16 = g16 + lane
                wv = [wts[kk, pl.ds(g16, _L)] for kk in range(4)]
                for c in range(_L):
                    cv = jnp.full((_L,), c, jnp.int32)
                    acc = (plsc.load_gather(s00, [rows16, cv]) * wv[0]
                           + plsc.load_gather(s01, [rows16, cv]) * wv[1]
                           + plsc.load_gather(s10, [rows16, cv]) * wv[2]
                           + plsc.load_gather(s11, [rows16, cv]) * wv[3])
                    plsc.store_scatter(outv, [rows16, cv + 32], acc)
            pltpu.sync_copy(outv, out_hbm.at[pl.ds(base, _B)])
            return 0

        lax.fori_loop(0, steps, step, 0)

    return k(ux, uy, g0f, t1)


def kernel(uv, g0, g1):
    c0, res0 = g0.shape[1], g0.shape[2]
    c1, res1 = g1.shape[1], g1.shape[2]
    n = uv.shape[0]
    g0f = g0.reshape(c0, res0 * res0)
    t1 = jnp.transpose(g1[0], (1, 2, 0)).reshape(res1 * res1, c1)
    ux = uv[:, 0] + 0.0
    uy = uv[:, 1] + 0.0
    return _feature_level_sc(ux, uy, g0f, t1, n, res0, res1, c0, c1)


# speedup vs baseline: 1.2896x; 1.2896x over previous
import functools

import jax
import jax.numpy as jnp
from jax import lax
from jax.experimental import pallas as pl
from jax.experimental.pallas import tpu as pltpu
from jax.experimental.pallas import tpu_sc as plsc

_L = 16
_B = 128
_K = 2048


def _floor_i32(x):
    t = x.astype(jnp.int32)
    return jnp.where(x < t.astype(jnp.float32), t - 1, t)


def _feature_level_sc(ux, uy, g0f, t1, n, res0, res1, c0, c1):
    nworkers = 32
    per_w = n // nworkers
    steps = per_w // _B
    cout = 4 * c0 + c1
    npos = res0 * res0
    plane = npos
    pos_per_tile = npos // 16
    blocks = pos_per_tile // _K

    mesh = plsc.VectorSubcoreMesh(core_axis_name="c", subcore_axis_name="s")

    @functools.partial(
        pl.kernel,
        mesh=mesh,
        compiler_params=pltpu.CompilerParams(use_tc_tiling_on_sc=False,
                                             needs_layout_passes=False),
        out_type=jax.ShapeDtypeStruct((n, cout), jnp.float32),
        scratch_types=[
            pltpu.VMEM((_B,), jnp.float32),
            pltpu.VMEM((_B,), jnp.float32),
            pltpu.VMEM((8, _B), jnp.int32),
            pltpu.VMEM((4, _B), jnp.float32),
            pltpu.VMEM((_B, 8), jnp.float32),
            pltpu.VMEM((_B, 8), jnp.float32),
            pltpu.VMEM((_B, 8), jnp.float32),
            pltpu.VMEM((_B, 8), jnp.float32),
            pltpu.VMEM((_B, 16), jnp.float32),
            pltpu.VMEM((_B, 16), jnp.float32),
            pltpu.VMEM((_B, 16), jnp.float32),
            pltpu.VMEM((_B, 16), jnp.float32),
            pltpu.VMEM((_B, 48), jnp.float32),
            pltpu.VMEM((8, _K), jnp.float32),
            pltpu.VMEM((8, _K), jnp.float32),
            pltpu.VMEM((_K, 8), jnp.float32),
            pltpu.VMEM((_K, 8), jnp.float32),
            pltpu.HBM((2, npos, 8), jnp.float32),
            pltpu.SemaphoreType.DMA,
            pltpu.SemaphoreType.DMA,
            pltpu.SemaphoreType.DMA,
            pltpu.SemaphoreType.DMA,
            pltpu.SemaphoreType.DMA,
        ],
    )
    def k(ux_hbm, uy_hbm, g0_hbm, t1_hbm, out_hbm,
          uxv, uyv, idx, wts, c00, c01, c10, c11, s00, s01, s10, s11,
          outv, tp0, tp1, to0, to1, tbl, sem, isem0, isem1, osem0, osem1):
        sid = lax.axis_index("s")
        cc = lax.axis_index("c")
        wid = sid * 2 + cc

        def in_src(blk):
            return g0_hbm.at[:, pl.ds(sid * pos_per_tile + blk * _K, _K)]

        def out_dst(blk):
            return tbl.at[cc, pl.ds(sid * pos_per_tile + blk * _K, _K)]

        pltpu.async_copy(in_src(0), tp0, isem0)
        pltpu.async_copy(in_src(1), tp1, isem1)

        def tpair(b2, _):
            for par, (tpb, tob, isem, osem) in enumerate(
                    ((tp0, to0, isem0, osem0), (tp1, to1, isem1, osem1))):
                blk = b2 * 2 + par
                pltpu.make_async_copy(in_src(blk), tpb, isem).wait()

                @pl.when(blk >= 2)
                def _():
                    pltpu.make_async_copy(tob, out_dst(blk - 2), osem).wait()

                def shuffle(q, _):
                    lane = lax.iota(jnp.int32, _L)
                    drows = q * _L + lane
                    vs = [tpb[c, pl.ds(q * _L, _L)] for c in range(8)]
                    for c in range(8):
                        cv = jnp.full((_L,), c, jnp.int32)
                        plsc.store_scatter(tob, [drows, cv], vs[c])
                    return 0

                lax.fori_loop(0, _K // _L, shuffle, 0)
                pltpu.async_copy(tob, out_dst(blk), osem)

                @pl.when(blk + 2 < blocks)
                def _():
                    pltpu.async_copy(in_src(blk + 2), tpb, isem)
            return 0

        lax.fori_loop(0, blocks // 2, tpair, 0)
        pltpu.make_async_copy(to0, out_dst(blocks - 2), osem0).wait()
        pltpu.make_async_copy(to1, out_dst(blocks - 1), osem1).wait()
        plsc.subcore_barrier()

        def step(st, _):
            base = wid * per_w + st * _B
            pltpu.sync_copy(ux_hbm.at[pl.ds(base, _B)], uxv)
            pltpu.sync_copy(uy_hbm.at[pl.ds(base, _B)], uyv)

            for g in range(_B // _L):
                sl = pl.ds(g * _L, _L)
                x = uxv[sl]
                y = uyv[sl]
                fx = _floor_i32(x * res0 - 0.5)
                fy = _floor_i32(y * res0 - 0.5)
                x0 = jnp.clip(fx, 0, res0 - 2)
                y0 = jnp.clip(fy, 0, res0 - 2)
                b00 = y0 * res0 + x0
                lane = lax.iota(jnp.int32, _L)
                r_a, r_b = (0, 2) if g < 4 else (1, 3)
                posv = 2 * ((g % 4) * _L) + 2 * lane
                plsc.store_scatter(idx.at[r_a], [posv], b00)
                plsc.store_scatter(idx.at[r_a], [posv + 1], b00 + 1)
                plsc.store_scatter(idx.at[r_b], [posv], b00 + res0)
                plsc.store_scatter(idx.at[r_b], [posv + 1], b00 + res0 + 1)
                qx = x * res1 - 0.5
                qy = y * res1 - 0.5
                ix0 = _floor_i32(qx)
                iy0 = _floor_i32(qy)
                wx1 = qx - ix0.astype(jnp.float32)
                wy1 = qy - iy0.astype(jnp.float32)
                wx0 = 1.0 - wx1
                wy0 = 1.0 - wy1
                wx0 = jnp.where(ix0 >= 0, wx0, 0.0)
                wy0 = jnp.where(iy0 >= 0, wy0, 0.0)
                wx1 = jnp.where(ix0 + 1 <= res1 - 1, wx1, 0.0)
                wy1 = jnp.where(iy0 + 1 <= res1 - 1, wy1, 0.0)
                jx0 = jnp.maximum(ix0, 0)
                jy0 = jnp.maximum(iy0, 0)
                jx1 = jnp.minimum(ix0 + 1, res1 - 1)
                jy1 = jnp.minimum(iy0 + 1, res1 - 1)
                idx[4, sl] = jy0 * res1 + jx0
                idx[5, sl] = jy0 * res1 + jx1
                idx[6, sl] = jy1 * res1 + jx0
                idx[7, sl] = jy1 * res1 + jx1
                wts[0, sl] = wy0 * wx0
                wts[1, sl] = wy0 * wx1
                wts[2, sl] = wy1 * wx0
                wts[3, sl] = wy1 * wx1

            cps = [
                pltpu.async_copy(tbl.at[cc].at[idx.at[0]], c00, sem),
                pltpu.async_copy(tbl.at[cc].at[idx.at[1]], c01, sem),
                pltpu.async_copy(tbl.at[cc].at[idx.at[2]], c10, sem),
                pltpu.async_copy(tbl.at[cc].at[idx.at[3]], c11, sem),
                pltpu.async_copy(t1_hbm.at[idx.at[4]], s00, sem),
                pltpu.async_copy(t1_hbm.at[idx.at[5]], s01, sem),
                pltpu.async_copy(t1_hbm.at[idx.at[6]], s10, sem),
                pltpu.async_copy(t1_hbm.at[idx.at[7]], s11, sem),
            ]
            for cp in cps:
                cp.wait()

            for g in range(_B // _L):
                g16 = g * _L
                lane = lax.iota(jnp.int32, _L)
                colsrc = lane & 7
                ca, cb = (c00, c10) if g < 4 else (c01, c11)
                for h in range(2):
                    vals = []
                    for t8 in range(8):
                        t = h * 8 + t8
                        il = (g % 4) * _L + t
                        prow = 2 * il + (lane >> 3)
                        vals.append(plsc.load_gather(ca, [prow, colsrc]))
                        vals.append(plsc.load_gather(cb, [prow, colsrc]))
                    for t8 in range(8):
                        t = h * 8 + t8
                        outv[g16 + t, 0:16] = vals[2 * t8]
                        outv[g16 + t, 16:32] = vals[2 * t8 + 1]
                rows16 = g16 + lane
                wv = [wts[kk, pl.ds(g16, _L)] for kk in range(4)]
                accs = []
                for c in range(_L):
                    cv = jnp.full((_L,), c, jnp.int32)
                    accs.append(
                        plsc.load_gather(s00, [rows16, cv]) * wv[0]
                        + plsc.load_gather(s01, [rows16, cv]) * wv[1]
                        + plsc.load_gather(s10, [rows16, cv]) * wv[2]
                        + plsc.load_gather(s11, [rows16, cv]) * wv[3])
                for c in range(_L):
                    cv = jnp.full((_L,), c, jnp.int32)
                    plsc.store_scatter(outv, [rows16, cv + 32], accs[c])
            pltpu.sync_copy(outv, out_hbm.at[pl.ds(base, _B)])
            return 0

        lax.fori_loop(0, steps, step, 0)

    return k(ux, uy, g0f, t1)


def kernel(uv, g0, g1):
    c0, res0 = g0.shape[1], g0.shape[2]
    c1, res1 = g1.shape[1], g1.shape[2]
    n = uv.shape[0]
    g0f = g0.reshape(c0, res0 * res0)
    t1 = jnp.transpose(g1[0], (1, 2, 0)).reshape(res1 * res1, c1)
    ux = uv[:, 0] + 0.0
    uy = uv[:, 1] + 0.0
    return _feature_level_sc(ux, uy, g0f, t1, n, res0, res1, c0, c1)


# speedup vs baseline: 1.4222x; 1.1028x over previous
import functools

import jax
import jax.numpy as jnp
from jax import lax
from jax.experimental import pallas as pl
from jax.experimental.pallas import tpu as pltpu
from jax.experimental.pallas import tpu_sc as plsc

_L = 16
_B = 128
_K = 1024


def _floor_i32(x):
    t = x.astype(jnp.int32)
    return jnp.where(x < t.astype(jnp.float32), t - 1, t)


def _feature_level_sc(ux, uy, g0f, t1, n, res0, res1, c0, c1):
    nworkers = 32
    per_w = n // nworkers
    steps = per_w // _B
    cout = 4 * c0 + c1
    npos = res0 * res0
    pos_per_tile = npos // 16
    blocks = pos_per_tile // _K

    mesh = plsc.VectorSubcoreMesh(core_axis_name="c", subcore_axis_name="s")

    dbl = lambda t: (t, t)

    @functools.partial(
        pl.kernel,
        mesh=mesh,
        compiler_params=pltpu.CompilerParams(use_tc_tiling_on_sc=False,
                                             needs_layout_passes=False),
        out_type=jax.ShapeDtypeStruct((n, cout), jnp.float32),
        scratch_types=[
            pltpu.VMEM((per_w,), jnp.float32),
            pltpu.VMEM((per_w,), jnp.float32),
            dbl(pltpu.VMEM((8, _B), jnp.int32)),
            dbl(pltpu.VMEM((4, _B), jnp.float32)),
            dbl(pltpu.VMEM((_B, 8), jnp.float32)),
            dbl(pltpu.VMEM((_B, 8), jnp.float32)),
            dbl(pltpu.VMEM((_B, 8), jnp.float32)),
            dbl(pltpu.VMEM((_B, 8), jnp.float32)),
            dbl(pltpu.VMEM((_B, 16), jnp.float32)),
            dbl(pltpu.VMEM((_B, 16), jnp.float32)),
            dbl(pltpu.VMEM((_B, 16), jnp.float32)),
            dbl(pltpu.VMEM((_B, 16), jnp.float32)),
            dbl(pltpu.VMEM((_B, 48), jnp.float32)),
            dbl(pltpu.SemaphoreType.DMA),
            dbl(pltpu.SemaphoreType.DMA),
            pltpu.VMEM((8, _K), jnp.float32),
            pltpu.VMEM((8, _K), jnp.float32),
            pltpu.VMEM((_K, 8), jnp.float32),
            pltpu.VMEM((_K, 8), jnp.float32),
            pltpu.HBM((2, npos, 8), jnp.float32),
            pltpu.SemaphoreType.DMA,
            pltpu.SemaphoreType.DMA,
            pltpu.SemaphoreType.DMA,
            pltpu.SemaphoreType.DMA,
            pltpu.SemaphoreType.DMA,
        ],
    )
    def k(ux_hbm, uy_hbm, g0_hbm, t1_hbm, out_hbm,
          uxall, uyall, idx2, wts2, ca_lo2, ca_hi2, cb_lo2, cb_hi2,
          s002, s012, s102, s112, outv2, gsem2, posem2,
          tp0, tp1, to0, to1, tbl, uvsem, isem0, isem1, osem0, osem1):
        sid = lax.axis_index("s")
        cc = lax.axis_index("c")
        wid = sid * 2 + cc
        wbase = wid * per_w

        pltpu.async_copy(ux_hbm.at[pl.ds(wbase, per_w)], uxall, uvsem)
        pltpu.async_copy(uy_hbm.at[pl.ds(wbase, per_w)], uyall, uvsem)

        def in_src(blk):
            return g0_hbm.at[:, pl.ds(sid * pos_per_tile + blk * _K, _K)]

        def out_dst(blk):
            return tbl.at[cc, pl.ds(sid * pos_per_tile + blk * _K, _K)]

        pltpu.async_copy(in_src(0), tp0, isem0)
        pltpu.async_copy(in_src(1), tp1, isem1)

        def tpair(b2, _):
            for par, (tpb, tob, isem, osem) in enumerate(
                    ((tp0, to0, isem0, osem0), (tp1, to1, isem1, osem1))):
                blk = b2 * 2 + par
                pltpu.make_async_copy(in_src(blk), tpb, isem).wait()

                @pl.when(blk >= 2)
                def _():
                    pltpu.make_async_copy(tob, out_dst(blk - 2), osem).wait()

                def shuffle(q, _):
                    lane = lax.iota(jnp.int32, _L)
                    drows = q * _L + lane
                    vs = [tpb[c, pl.ds(q * _L, _L)] for c in range(8)]
                    for c in range(8):
                        cv = jnp.full((_L,), c, jnp.int32)
                        plsc.store_scatter(tob, [drows, cv], vs[c])
                    return 0

                lax.fori_loop(0, _K // _L, shuffle, 0)
                pltpu.async_copy(tob, out_dst(blk), osem)

                @pl.when(blk + 2 < blocks)
                def _():
                    pltpu.async_copy(in_src(blk + 2), tpb, isem)
            return 0

        lax.fori_loop(0, blocks // 2, tpair, 0)
        pltpu.make_async_copy(to0, out_dst(blocks - 2), osem0).wait()
        pltpu.make_async_copy(to1, out_dst(blocks - 1), osem1).wait()
        plsc.subcore_barrier()
        pltpu.make_async_copy(ux_hbm.at[pl.ds(wbase, per_w)], uxall,
                              uvsem).wait()
        pltpu.make_async_copy(uy_hbm.at[pl.ds(wbase, per_w)], uyall,
                              uvsem).wait()

        sets = tuple(
            dict(idx=idx2[p], wts=wts2[p], ca_lo=ca_lo2[p], ca_hi=ca_hi2[p],
                 cb_lo=cb_lo2[p], cb_hi=cb_hi2[p], s00=s002[p], s01=s012[p],
                 s10=s102[p], s11=s112[p], outv=outv2[p], gsem=gsem2[p],
                 posem=posem2[p])
            for p in range(2))

        def gather_cps(bs):
            tb = tbl.at[cc]
            idx = bs["idx"]
            return [
                pltpu.make_async_copy(tb.at[idx.at[0]], bs["ca_lo"], bs["gsem"]),
                pltpu.make_async_copy(tb.at[idx.at[1]], bs["ca_hi"], bs["gsem"]),
                pltpu.make_async_copy(tb.at[idx.at[2]], bs["cb_lo"], bs["gsem"]),
                pltpu.make_async_copy(tb.at[idx.at[3]], bs["cb_hi"], bs["gsem"]),
                pltpu.make_async_copy(t1_hbm.at[idx.at[4]], bs["s00"], bs["gsem"]),
                pltpu.make_async_copy(t1_hbm.at[idx.at[5]], bs["s01"], bs["gsem"]),
                pltpu.make_async_copy(t1_hbm.at[idx.at[6]], bs["s10"], bs["gsem"]),
                pltpu.make_async_copy(t1_hbm.at[idx.at[7]], bs["s11"], bs["gsem"]),
            ]

        def prep(st, bs):
            idx = bs["idx"]
            wts = bs["wts"]
            soff = st * _B
            for g in range(_B // _L):
                sl = pl.ds(soff + g * _L, _L)
                x = uxall[sl]
                y = uyall[sl]
                fx = _floor_i32(x * res0 - 0.5)
                fy = _floor_i32(y * res0 - 0.5)
                x0 = jnp.clip(fx, 0, res0 - 2)
                y0 = jnp.clip(fy, 0, res0 - 2)
                b00 = y0 * res0 + x0
                lane = lax.iota(jnp.int32, _L)
                r_a, r_b = (0, 2) if g < 4 else (1, 3)
                posv = 2 * ((g % 4) * _L) + 2 * lane
                plsc.store_scatter(idx.at[r_a], [posv], b00)
                plsc.store_scatter(idx.at[r_a], [posv + 1], b00 + 1)
                plsc.store_scatter(idx.at[r_b], [posv], b00 + res0)
                plsc.store_scatter(idx.at[r_b], [posv + 1], b00 + res0 + 1)
                gsl = pl.ds(g * _L, _L)
                qx = x * res1 - 0.5
                qy = y * res1 - 0.5
                ix0 = _floor_i32(qx)
                iy0 = _floor_i32(qy)
                wx1 = qx - ix0.astype(jnp.float32)
                wy1 = qy - iy0.astype(jnp.float32)
                wx0 = 1.0 - wx1
                wy0 = 1.0 - wy1
                wx0 = jnp.where(ix0 >= 0, wx0, 0.0)
                wy0 = jnp.where(iy0 >= 0, wy0, 0.0)
                wx1 = jnp.where(ix0 + 1 <= res1 - 1, wx1, 0.0)
                wy1 = jnp.where(iy0 + 1 <= res1 - 1, wy1, 0.0)
                jx0 = jnp.maximum(ix0, 0)
                jy0 = jnp.maximum(iy0, 0)
                jx1 = jnp.minimum(ix0 + 1, res1 - 1)
                jy1 = jnp.minimum(iy0 + 1, res1 - 1)
                idx[4, gsl] = jy0 * res1 + jx0
                idx[5, gsl] = jy0 * res1 + jx1
                idx[6, gsl] = jy1 * res1 + jx0
                idx[7, gsl] = jy1 * res1 + jx1
                wts[0, gsl] = wy0 * wx0
                wts[1, gsl] = wy0 * wx1
                wts[2, gsl] = wy1 * wx0
                wts[3, gsl] = wy1 * wx1
            for cp in gather_cps(bs):
                cp.start()

        def out_cp(st, bs):
            return pltpu.make_async_copy(
                bs["outv"], out_hbm.at[pl.ds(wbase + st * _B, _B)], bs["posem"])

        def assemble(st, bs):
            outv = bs["outv"]
            wts = bs["wts"]
            for g in range(_B // _L):
                g16 = g * _L
                lane = lax.iota(jnp.int32, _L)
                colsrc = lane & 7
                ca, cb = ((bs["ca_lo"], bs["cb_lo"]) if g < 4
                          else (bs["ca_hi"], bs["cb_hi"]))
                for h in range(2):
                    vals = []
                    for t8 in range(8):
                        t = h * 8 + t8
                        il = (g % 4) * _L + t
                        prow = 2 * il + (lane >> 3)
                        vals.append(plsc.load_gather(ca, [prow, colsrc]))
                        vals.append(plsc.load_gather(cb, [prow, colsrc]))
                    for t8 in range(8):
                        t = h * 8 + t8
                        outv[g16 + t, 0:16] = vals[2 * t8]
                        outv[g16 + t, 16:32] = vals[2 * t8 + 1]
                rows16 = g16 + lane
                wv = [wts[kk, pl.ds(g16, _L)] for kk in range(4)]
                accs = []
                for c in range(_L):
                    cv = jnp.full((_L,), c, jnp.int32)
                    accs.append(
                        plsc.load_gather(bs["s00"], [rows16, cv]) * wv[0]
                        + plsc.load_gather(bs["s01"], [rows16, cv]) * wv[1]
                        + plsc.load_gather(bs["s10"], [rows16, cv]) * wv[2]
                        + plsc.load_gather(bs["s11"], [rows16, cv]) * wv[3])
                for c in range(_L):
                    cv = jnp.full((_L,), c, jnp.int32)
                    plsc.store_scatter(outv, [rows16, cv + 32], accs[c])
            out_cp(st, bs).start()

        prep(0, sets[0])

        def spair(s2, _):
            for par in range(2):
                st = s2 * 2 + par
                bs = sets[par]
                for cp in gather_cps(bs):
                    cp.wait()

                @pl.when(st + 1 < steps)
                def _():
                    prep(st + 1, sets[1 - par])

                @pl.when(st >= 2)
                def _():
                    out_cp(st - 2, bs).wait()

                assemble(st, bs)
            return 0

        lax.fori_loop(0, steps // 2, spair, 0)
        out_cp(steps - 2, sets[0]).wait()
        out_cp(steps - 1, sets[1]).wait()

    return k(ux, uy, g0f, t1)


def kernel(uv, g0, g1):
    c0, res0 = g0.shape[1], g0.shape[2]
    c1, res1 = g1.shape[1], g1.shape[2]
    n = uv.shape[0]
    g0f = g0.reshape(c0, res0 * res0)
    t1 = jnp.transpose(g1[0], (1, 2, 0)).reshape(res1 * res1, c1)
    ux = uv[:, 0] + 0.0
    uy = uv[:, 1] + 0.0
    return _feature_level_sc(ux, uy, g0f, t1, n, res0, res1, c0, c1)


# speedup vs baseline: 1.7051x; 1.1989x over previous
import functools

import jax
import jax.numpy as jnp
from jax import lax
from jax.experimental import pallas as pl
from jax.experimental.pallas import tpu as pltpu
from jax.experimental.pallas import tpu_sc as plsc

_L = 16
_B = 128
_K = 1024


def _floor_i32(x):
    t = x.astype(jnp.int32)
    return jnp.where(x < t.astype(jnp.float32), t - 1, t)


def _feature_level_sc(ux, uy, g0f, t1, n, res0, res1, c0, c1):
    nworkers = 32
    per_w = n // nworkers
    steps = per_w // _B
    cout = 4 * c0 + c1
    npos = res0 * res0
    pos_per_tile = npos // 16
    blocks = pos_per_tile // _K

    mesh = plsc.VectorSubcoreMesh(core_axis_name="c", subcore_axis_name="s")

    tri = lambda t: (t, t, t)

    @functools.partial(
        pl.kernel,
        mesh=mesh,
        compiler_params=pltpu.CompilerParams(use_tc_tiling_on_sc=False,
                                             needs_layout_passes=False),
        out_type=jax.ShapeDtypeStruct((n, cout), jnp.float32),
        scratch_types=[
            pltpu.VMEM((per_w,), jnp.float32),
            pltpu.VMEM((per_w,), jnp.float32),
            tri(pltpu.VMEM((8, _B), jnp.int32)),
            tri(pltpu.VMEM((4, _B), jnp.float32)),
            tri(pltpu.VMEM((_B, 8), jnp.float32)),
            tri(pltpu.VMEM((_B, 8), jnp.float32)),
            tri(pltpu.VMEM((_B, 8), jnp.float32)),
            tri(pltpu.VMEM((_B, 8), jnp.float32)),
            tri(pltpu.VMEM((_B, 16), jnp.float32)),
            tri(pltpu.VMEM((_B, 16), jnp.float32)),
            tri(pltpu.VMEM((_B, 16), jnp.float32)),
            tri(pltpu.VMEM((_B, 16), jnp.float32)),
            tri(pltpu.VMEM((_B, 16), jnp.float32)),
            tri(pltpu.SemaphoreType.DMA),
            tri(pltpu.SemaphoreType.DMA),
            pltpu.VMEM((8, _K), jnp.float32),
            pltpu.VMEM((8, _K), jnp.float32),
            pltpu.VMEM((_K, 8), jnp.float32),
            pltpu.VMEM((_K, 8), jnp.float32),
            pltpu.HBM((2, npos, 8), jnp.float32),
            pltpu.SemaphoreType.DMA,
            pltpu.SemaphoreType.DMA,
            pltpu.SemaphoreType.DMA,
            pltpu.SemaphoreType.DMA,
            pltpu.SemaphoreType.DMA,
        ],
    )
    def k(ux_hbm, uy_hbm, g0_hbm, t1_hbm, out_hbm,
          uxall, uyall, idx2, wts2,
          c002, c012, c102, c112,
          s002, s012, s102, s112, f1v2, gsem2, posem2,
          tp0, tp1, to0, to1, tbl, uvsem, isem0, isem1, osem0, osem1):
        sid = lax.axis_index("s")
        cc = lax.axis_index("c")
        wid = sid * 2 + cc
        wbase = wid * per_w

        pltpu.async_copy(ux_hbm.at[pl.ds(wbase, per_w)], uxall, uvsem)
        pltpu.async_copy(uy_hbm.at[pl.ds(wbase, per_w)], uyall, uvsem)

        def in_src(blk):
            return g0_hbm.at[:, pl.ds(sid * pos_per_tile + blk * _K, _K)]

        def out_dst(blk):
            return tbl.at[cc, pl.ds(sid * pos_per_tile + blk * _K, _K)]

        pltpu.async_copy(in_src(0), tp0, isem0)
        pltpu.async_copy(in_src(1), tp1, isem1)

        def tpair(b2, _):
            for par, (tpb, tob, isem, osem) in enumerate(
                    ((tp0, to0, isem0, osem0), (tp1, to1, isem1, osem1))):
                blk = b2 * 2 + par
                pltpu.make_async_copy(in_src(blk), tpb, isem).wait()

                @pl.when(blk >= 2)
                def _():
                    pltpu.make_async_copy(tob, out_dst(blk - 2), osem).wait()

                def shuffle(q, _):
                    lane = lax.iota(jnp.int32, _L)
                    drows = q * _L + lane
                    vs = [tpb[c, pl.ds(q * _L, _L)] for c in range(8)]
                    for c in range(8):
                        cv = jnp.full((_L,), c, jnp.int32)
                        plsc.store_scatter(tob, [drows, cv], vs[c])
                    return 0

                lax.fori_loop(0, _K // _L, shuffle, 0)
                pltpu.async_copy(tob, out_dst(blk), osem)

                @pl.when(blk + 2 < blocks)
                def _():
                    pltpu.async_copy(in_src(blk + 2), tpb, isem)
            return 0

        lax.fori_loop(0, blocks // 2, tpair, 0)
        pltpu.make_async_copy(to0, out_dst(blocks - 2), osem0).wait()
        pltpu.make_async_copy(to1, out_dst(blocks - 1), osem1).wait()
        plsc.subcore_barrier()
        pltpu.make_async_copy(ux_hbm.at[pl.ds(wbase, per_w)], uxall,
                              uvsem).wait()
        pltpu.make_async_copy(uy_hbm.at[pl.ds(wbase, per_w)], uyall,
                              uvsem).wait()

        sets = tuple(
            dict(idx=idx2[p], wts=wts2[p],
                 c00=c002[p], c01=c012[p], c10=c102[p], c11=c112[p],
                 s00=s002[p], s01=s012[p],
                 s10=s102[p], s11=s112[p], f1v=f1v2[p], gsem=gsem2[p],
                 posem=posem2[p])
            for p in range(3))

        def gather_cps(bs):
            tb = tbl.at[cc]
            idx = bs["idx"]
            return [
                pltpu.make_async_copy(tb.at[idx.at[0]], bs["c00"], bs["gsem"]),
                pltpu.make_async_copy(tb.at[idx.at[1]], bs["c01"], bs["gsem"]),
                pltpu.make_async_copy(tb.at[idx.at[2]], bs["c10"], bs["gsem"]),
                pltpu.make_async_copy(tb.at[idx.at[3]], bs["c11"], bs["gsem"]),
                pltpu.make_async_copy(t1_hbm.at[idx.at[4]], bs["s00"], bs["gsem"]),
                pltpu.make_async_copy(t1_hbm.at[idx.at[5]], bs["s01"], bs["gsem"]),
                pltpu.make_async_copy(t1_hbm.at[idx.at[6]], bs["s10"], bs["gsem"]),
                pltpu.make_async_copy(t1_hbm.at[idx.at[7]], bs["s11"], bs["gsem"]),
            ]

        def out_cps(st, bs):
            al = pl.ds(wbase + st * _B, _B)
            sem = bs["posem"]
            return [
                pltpu.make_async_copy(bs["c00"], out_hbm.at[al, 0:8], sem),
                pltpu.make_async_copy(bs["c01"], out_hbm.at[al, 8:16], sem),
                pltpu.make_async_copy(bs["c10"], out_hbm.at[al, 16:24], sem),
                pltpu.make_async_copy(bs["c11"], out_hbm.at[al, 24:32], sem),
                pltpu.make_async_copy(bs["f1v"], out_hbm.at[al, 32:48], sem),
            ]

        def prep(st, bs):
            idx = bs["idx"]
            wts = bs["wts"]
            soff = st * _B
            for g in range(_B // _L):
                sl = pl.ds(soff + g * _L, _L)
                x = uxall[sl]
                y = uyall[sl]
                fx = _floor_i32(x * res0 - 0.5)
                fy = _floor_i32(y * res0 - 0.5)
                x0 = jnp.clip(fx, 0, res0 - 2)
                y0 = jnp.clip(fy, 0, res0 - 2)
                b00 = y0 * res0 + x0
                gsl0 = pl.ds(g * _L, _L)
                idx[0, gsl0] = b00
                idx[1, gsl0] = b00 + 1
                idx[2, gsl0] = b00 + res0
                idx[3, gsl0] = b00 + res0 + 1
                gsl = pl.ds(g * _L, _L)
                qx = x * res1 - 0.5
                qy = y * res1 - 0.5
                ix0 = _floor_i32(qx)
                iy0 = _floor_i32(qy)
                wx1 = qx - ix0.astype(jnp.float32)
                wy1 = qy - iy0.astype(jnp.float32)
                wx0 = 1.0 - wx1
                wy0 = 1.0 - wy1
                wx0 = jnp.where(ix0 >= 0, wx0, 0.0)
                wy0 = jnp.where(iy0 >= 0, wy0, 0.0)
                wx1 = jnp.where(ix0 + 1 <= res1 - 1, wx1, 0.0)
                wy1 = jnp.where(iy0 + 1 <= res1 - 1, wy1, 0.0)
                jx0 = jnp.maximum(ix0, 0)
                jy0 = jnp.maximum(iy0, 0)
                jx1 = jnp.minimum(ix0 + 1, res1 - 1)
                jy1 = jnp.minimum(iy0 + 1, res1 - 1)
                idx[4, gsl] = jy0 * res1 + jx0
                idx[5, gsl] = jy0 * res1 + jx1
                idx[6, gsl] = jy1 * res1 + jx0
                idx[7, gsl] = jy1 * res1 + jx1
                wts[0, gsl] = wy0 * wx0
                wts[1, gsl] = wy0 * wx1
                wts[2, gsl] = wy1 * wx0
                wts[3, gsl] = wy1 * wx1
            for cp in gather_cps(bs):
                cp.start()

        def assemble(st, bs):
            f1v = bs["f1v"]
            wts = bs["wts"]
            for g in range(_B // _L):
                g16 = g * _L
                lane = lax.iota(jnp.int32, _L)
                rows16 = g16 + lane
                wv = [wts[kk, pl.ds(g16, _L)] for kk in range(4)]
                accs = []
                for c in range(_L):
                    cv = jnp.full((_L,), c, jnp.int32)
                    accs.append(
                        plsc.load_gather(bs["s00"], [rows16, cv]) * wv[0]
                        + plsc.load_gather(bs["s01"], [rows16, cv]) * wv[1]
                        + plsc.load_gather(bs["s10"], [rows16, cv]) * wv[2]
                        + plsc.load_gather(bs["s11"], [rows16, cv]) * wv[3])
                for c in range(_L):
                    cv = jnp.full((_L,), c, jnp.int32)
                    plsc.store_scatter(f1v, [rows16, cv], accs[c])
            for cp in out_cps(st, bs):
                cp.start()

        def body(st, cur, nxt):
            bs = sets[cur]
            for cp in gather_cps(bs):
                cp.wait()

            @pl.when(st + 1 < steps)
            def _():
                @pl.when(st >= 2)
                def _():
                    for cp in out_cps(st - 2, sets[nxt]):
                        cp.wait()
                prep(st + 1, sets[nxt])

            assemble(st, bs)

        prep(0, sets[0])

        def striple(s3, _):
            for par in range(3):
                st = s3 * 3 + par
                body(st, par, (par + 1) % 3)
            return 0

        lax.fori_loop(0, steps // 3, striple, 0)
        st_tail = (steps // 3) * 3
        for st in range(st_tail, steps):
            body(st, st % 3, (st + 1) % 3)
        for st in (steps - 3, steps - 2, steps - 1):
            for cp in out_cps(st, sets[st % 3]):
                cp.wait()

    return k(ux, uy, g0f, t1)


def kernel(uv, g0, g1):
    c0, res0 = g0.shape[1], g0.shape[2]
    c1, res1 = g1.shape[1], g1.shape[2]
    n = uv.shape[0]
    g0f = g0.reshape(c0, res0 * res0)
    t1 = jnp.transpose(g1[0], (1, 2, 0)).reshape(res1 * res1, c1)
    ux = uv[:, 0] + 0.0
    uy = uv[:, 1] + 0.0
    return _feature_level_sc(ux, uy, g0f, t1, n, res0, res1, c0, c1)
